# baseline (speedup 1.0000x reference)
import functools

import jax
import jax.numpy as jnp
from jax import lax
from jax.experimental import pallas as pl
from jax.experimental.pallas import tpu as pltpu
from jax.experimental.pallas import tpu_sc as plsc

N_EDGES = 640000
EMB_DIM = 128

_info = plsc.get_sparse_core_info()
NUM_CORES = _info.num_cores
NUM_SUBCORES = _info.num_subcores
NW = NUM_CORES * NUM_SUBCORES
B_PER_W = N_EDGES // NW
CHUNK = 80
GPS = 5
SUP = CHUNK * GPS
N_SUP = B_PER_W // SUP


def _sc_body(idx_hbm, table_hbm, out_hbm, table_sh, idx_v,
             buf0, buf1, g0, g1, w0, w1):
    sid = lax.axis_index("s")
    wid = sid * NUM_CORES + lax.axis_index("c")
    base = wid * B_PER_W

    @pl.when(sid == 0)
    def _():
        pltpu.sync_copy(table_hbm, table_sh)

    pltpu.sync_copy(idx_hbm.at[pl.ds(base, B_PER_W)], idx_v)
    plsc.subcore_barrier()

    def fire_gathers(s, buf, gsem):
        for i in range(GPS):
            pltpu.async_copy(
                table_sh.at[idx_v.at[pl.ds((s * GPS + i) * CHUNK, CHUNK)]],
                buf.at[pl.ds(i * CHUNK, CHUNK)], gsem)

    def wait_gathers(s, buf, gsem):
        for i in range(GPS):
            pltpu.make_async_copy(
                table_sh.at[idx_v.at[pl.ds((s * GPS + i) * CHUNK, CHUNK)]],
                buf.at[pl.ds(i * CHUNK, CHUNK)], gsem).wait()

    def write(s, buf, wsem):
        pltpu.make_async_copy(
            buf, out_hbm.at[pl.ds(base + s * SUP, SUP)], wsem).start()

    def wait_write(s, buf, wsem):
        pltpu.make_async_copy(
            buf, out_hbm.at[pl.ds(base + s * SUP, SUP)], wsem).wait()

    fire_gathers(0, buf0, g0)

    def body(k, _):
        s0 = 2 * k
        s1 = 2 * k + 1
        wait_gathers(s0, buf0, g0)
        write(s0, buf0, w0)

        @pl.when(k > 0)
        def _():
            wait_write(s1 - 2, buf1, w1)

        fire_gathers(s1, buf1, g1)
        wait_gathers(s1, buf1, g1)
        write(s1, buf1, w1)
        wait_write(s0, buf0, w0)

        @pl.when(s0 + 2 < N_SUP)
        def _():
            fire_gathers(s0 + 2, buf0, g0)

        return 0

    lax.fori_loop(0, N_SUP // 2, body, 0)
    wait_write(N_SUP - 1, buf1, w1)


@functools.partial(jax.jit, static_argnames=())
def _sc_lookup(idx, table):
    mesh = plsc.VectorSubcoreMesh(core_axis_name="c", subcore_axis_name="s")
    f = pl.kernel(
        _sc_body,
        out_type=jax.ShapeDtypeStruct((N_EDGES, EMB_DIM), jnp.float32),
        mesh=mesh,
        scratch_types=[
            pltpu.VMEM_SHARED((14, EMB_DIM), jnp.float32),
            pltpu.VMEM((B_PER_W,), jnp.int32),
            pltpu.VMEM((SUP, EMB_DIM), jnp.float32),
            pltpu.VMEM((SUP, EMB_DIM), jnp.float32),
            pltpu.SemaphoreType.DMA,
            pltpu.SemaphoreType.DMA,
            pltpu.SemaphoreType.DMA,
            pltpu.SemaphoreType.DMA,
        ],
    )
    return f(idx, table)


def kernel(hedge_attr, emb_weight):
    return _sc_lookup(hedge_attr.astype(jnp.int32), emb_weight)
